# baseline (speedup 1.0000x reference)
import functools
import math

import jax
import jax.numpy as jnp
from jax import lax
from jax.experimental import pallas as pl

_IM = 64
_KBLK = 4096
_TWO_PI = 6.283185307179586

_COS_C = [(-1.0) ** j * (2.0 * math.pi) ** (2 * j) / math.factorial(2 * j)
          for j in range(6)]
_SIN_C = [(-1.0) ** j * (2.0 * math.pi) ** (2 * j + 1) / math.factorial(2 * j + 1)
          for j in range(5)]


def _sincos_cycles(ph):
    n = jnp.round(4.0 * ph)
    z = ph - 0.25 * n
    y = z * z
    cp = _COS_C[5]
    for coef in (_COS_C[4], _COS_C[3], _COS_C[2], _COS_C[1], _COS_C[0]):
        cp = cp * y + coef
    sp = _SIN_C[4]
    for coef in (_SIN_C[3], _SIN_C[2], _SIN_C[1], _SIN_C[0]):
        sp = sp * y + coef
    sp = sp * z
    i = n.astype(jnp.int32)
    bit0 = (i & 1) == 1
    c_sign = ((i ^ (i >> 1)) & 1) << 31
    s_sign = ((i >> 1) & 1) << 31
    c_abs = jnp.where(bit0, sp, cp)
    s_abs = jnp.where(bit0, cp, sp)
    c = lax.bitcast_convert_type(
        lax.bitcast_convert_type(c_abs, jnp.int32) ^ c_sign, jnp.float32)
    s = lax.bitcast_convert_type(
        lax.bitcast_convert_type(s_abs, jnp.int32) ^ s_sign, jnp.float32)
    return c, s


def _nufft_block_kernel(trj_ref, img_t_ref, outr_ref, outi_ref, *, n_coils,
                        mxu_precision):
    kblk = trj_ref.shape[0]
    trj = trj_ref[...]
    trj = trj.astype(jnp.bfloat16).astype(jnp.float32)
    kx = trj[:, 0:1]
    ky = trj[:, 1:2]

    col = lax.broadcasted_iota(jnp.int32, (1, 2 * _IM), 1)
    rv2 = ((col % _IM) - (_IM // 2)).astype(jnp.float32) * (1.0 / _IM)
    kxy = jnp.where(col < _IM, kx, ky)
    ph = kxy * rv2
    c, s = _sincos_cycles(ph)
    cx = c[:, :_IM]
    sx = s[:, :_IM]
    cysy = jnp.concatenate([c[:, _IM:], s[:, _IM:]], axis=0)

    img_t = img_t_ref[...]
    dot = functools.partial(jnp.dot, preferred_element_type=jnp.float32,
                            precision=mxu_precision)
    st = dot(cysy, img_t)
    t_r = st[:kblk]
    t_n = st[kblk:]

    cxt = jnp.concatenate([cx] * n_coils, axis=1)
    sxt = jnp.concatenate([sx] * n_coils, axis=1)
    m_r = cxt * t_r - sxt * t_n
    m_n = cxt * t_n + sxt * t_r

    jj = lax.broadcasted_iota(jnp.int32, (n_coils * _IM, n_coils), 0)
    bb = lax.broadcasted_iota(jnp.int32, (n_coils * _IM, n_coils), 1)
    g = (jj // _IM == bb).astype(jnp.float32)
    m = jnp.concatenate([m_r, m_n], axis=0)
    res = lax.dot_general(g, m, (((0,), (1,)), ((), ())),
                          preferred_element_type=jnp.float32,
                          precision=mxu_precision)
    outr_ref[...] = res[:, :kblk]
    outi_ref[...] = -res[:, kblk:]


def kernel(img, trj):
    n = img.shape[0]
    n_coils = img.shape[1]
    n_k = trj.shape[1]
    img_t = (img.reshape(n_coils, _IM, _IM)
             .transpose(2, 0, 1)
             .reshape(_IM, n_coils * _IM))
    trj2 = trj.reshape(n_k, 2)

    body = functools.partial(_nufft_block_kernel, n_coils=n_coils,
                             mxu_precision=lax.Precision.DEFAULT)
    outr, outi = pl.pallas_call(
        body,
        grid=(n_k // _KBLK,),
        in_specs=[
            pl.BlockSpec((_KBLK, 2), lambda i: (i, 0)),
            pl.BlockSpec((_IM, n_coils * _IM), lambda i: (0, 0)),
        ],
        out_specs=[
            pl.BlockSpec((n_coils, _KBLK), lambda i: (0, i)),
            pl.BlockSpec((n_coils, _KBLK), lambda i: (0, i)),
        ],
        out_shape=[jax.ShapeDtypeStruct((n_coils, n_k), jnp.float32)] * 2,
    )(trj2, img_t)
    return lax.complex(outr, outi).reshape(n, n_coils, n_k)


# speedup vs baseline: 1.0144x; 1.0144x over previous
import functools
import math

import jax
import jax.numpy as jnp
from jax import lax
from jax.experimental import pallas as pl

_IM = 64
_KBLK = 2048
_TWO_PI = 6.283185307179586

_COS_C = [(-1.0) ** j * (2.0 * math.pi) ** (2 * j) / math.factorial(2 * j)
          for j in range(6)]
_SIN_C = [(-1.0) ** j * (2.0 * math.pi) ** (2 * j + 1) / math.factorial(2 * j + 1)
          for j in range(5)]


def _sincos_cycles(ph):
    n = jnp.round(4.0 * ph)
    z = ph - 0.25 * n
    y = z * z
    cp = _COS_C[5]
    for coef in (_COS_C[4], _COS_C[3], _COS_C[2], _COS_C[1], _COS_C[0]):
        cp = cp * y + coef
    sp = _SIN_C[4]
    for coef in (_SIN_C[3], _SIN_C[2], _SIN_C[1], _SIN_C[0]):
        sp = sp * y + coef
    sp = sp * z
    i = n.astype(jnp.int32)
    bit0 = (i & 1) == 1
    c_sign = ((i ^ (i >> 1)) & 1) << 31
    s_sign = ((i >> 1) & 1) << 31
    c_abs = jnp.where(bit0, sp, cp)
    s_abs = jnp.where(bit0, cp, sp)
    c = lax.bitcast_convert_type(
        lax.bitcast_convert_type(c_abs, jnp.int32) ^ c_sign, jnp.float32)
    s = lax.bitcast_convert_type(
        lax.bitcast_convert_type(s_abs, jnp.int32) ^ s_sign, jnp.float32)
    return c, s


def _nufft_block_kernel(trj_ref, img_t_ref, outr_ref, outi_ref, *, n_coils,
                        mxu_precision):
    kblk = trj_ref.shape[0]
    trj = trj_ref[...]
    trj = trj.astype(jnp.bfloat16).astype(jnp.float32)
    kx = trj[:, 0:1]
    ky = trj[:, 1:2]

    col = lax.broadcasted_iota(jnp.int32, (1, 2 * _IM), 1)
    rv2 = ((col % _IM) - (_IM // 2)).astype(jnp.float32) * (1.0 / _IM)
    kxy = jnp.where(col < _IM, kx, ky)
    ph = kxy * rv2
    c, s = _sincos_cycles(ph)
    cx = c[:, :_IM]
    sx = s[:, :_IM]
    cysy = jnp.concatenate([c[:, _IM:], s[:, _IM:]], axis=0)

    img_t = img_t_ref[...]
    dot = functools.partial(jnp.dot, preferred_element_type=jnp.float32,
                            precision=mxu_precision)
    st = dot(cysy, img_t)
    t_r = st[:kblk]
    t_n = st[kblk:]

    cxt = jnp.concatenate([cx] * n_coils, axis=1)
    sxt = jnp.concatenate([sx] * n_coils, axis=1)
    m_r = cxt * t_r - sxt * t_n
    m_n = cxt * t_n + sxt * t_r

    jj = lax.broadcasted_iota(jnp.int32, (n_coils * _IM, n_coils), 0)
    bb = lax.broadcasted_iota(jnp.int32, (n_coils * _IM, n_coils), 1)
    g = (jj // _IM == bb).astype(jnp.float32)
    m = jnp.concatenate([m_r, m_n], axis=0)
    res = lax.dot_general(g, m, (((0,), (1,)), ((), ())),
                          preferred_element_type=jnp.float32,
                          precision=mxu_precision)
    outr_ref[...] = res[:, :kblk]
    outi_ref[...] = -res[:, kblk:]


def kernel(img, trj):
    n = img.shape[0]
    n_coils = img.shape[1]
    n_k = trj.shape[1]
    img_t = (img.reshape(n_coils, _IM, _IM)
             .transpose(2, 0, 1)
             .reshape(_IM, n_coils * _IM))
    trj2 = trj.reshape(n_k, 2)

    body = functools.partial(_nufft_block_kernel, n_coils=n_coils,
                             mxu_precision=lax.Precision.DEFAULT)
    outr, outi = pl.pallas_call(
        body,
        grid=(n_k // _KBLK,),
        in_specs=[
            pl.BlockSpec((_KBLK, 2), lambda i: (i, 0)),
            pl.BlockSpec((_IM, n_coils * _IM), lambda i: (0, 0)),
        ],
        out_specs=[
            pl.BlockSpec((n_coils, _KBLK), lambda i: (0, i)),
            pl.BlockSpec((n_coils, _KBLK), lambda i: (0, i)),
        ],
        out_shape=[jax.ShapeDtypeStruct((n_coils, n_k), jnp.float32)] * 2,
    )(trj2, img_t)
    return lax.complex(outr, outi).reshape(n, n_coils, n_k)


# speedup vs baseline: 1.1089x; 1.0932x over previous
import functools
import math

import jax
import jax.numpy as jnp
from jax import lax
from jax.experimental import pallas as pl

_IM = 64
_KBLK = 2048
_TWO_PI = 6.283185307179586

_COS_C = [(-1.0) ** j * (2.0 * math.pi) ** (2 * j) / math.factorial(2 * j)
          for j in range(6)]
_SIN_C = [(-1.0) ** j * (2.0 * math.pi) ** (2 * j + 1) / math.factorial(2 * j + 1)
          for j in range(5)]


def _sincos_cycles(ph):
    n = jnp.round(4.0 * ph)
    z = ph - 0.25 * n
    y = z * z
    cp = _COS_C[5]
    for coef in (_COS_C[4], _COS_C[3], _COS_C[2], _COS_C[1], _COS_C[0]):
        cp = cp * y + coef
    sp = _SIN_C[4]
    for coef in (_SIN_C[3], _SIN_C[2], _SIN_C[1], _SIN_C[0]):
        sp = sp * y + coef
    sp = sp * z
    i = n.astype(jnp.int32)
    bit0 = (i & 1) == 1
    c_sign = ((i ^ (i >> 1)) & 1) << 31
    s_sign = ((i >> 1) & 1) << 31
    c_abs = jnp.where(bit0, sp, cp)
    s_abs = jnp.where(bit0, cp, sp)
    c = lax.bitcast_convert_type(
        lax.bitcast_convert_type(c_abs, jnp.int32) ^ c_sign, jnp.float32)
    s = lax.bitcast_convert_type(
        lax.bitcast_convert_type(s_abs, jnp.int32) ^ s_sign, jnp.float32)
    return c, s


def _nufft_block_kernel(trj_ref, img_t_ref, outr_ref, outi_ref, *, n_coils,
                        mxu_precision):
    kblk = trj_ref.shape[0]
    trj = trj_ref[...]
    trj = trj.astype(jnp.bfloat16).astype(jnp.float32)
    kx = trj[:, 0:1]
    ky = trj[:, 1:2]

    col = lax.broadcasted_iota(jnp.int32, (1, 2 * _IM), 1)
    rv2 = ((col % _IM) - (_IM // 2)).astype(jnp.float32) * (1.0 / _IM)
    kxy = jnp.where(col < _IM, kx, ky)
    ph = kxy * rv2
    c, s = _sincos_cycles(ph)
    cx = c[:, :_IM]
    sx = s[:, :_IM]
    cysy = jnp.concatenate([c[:, _IM:], s[:, _IM:]], axis=0)

    img2 = img_t_ref[...]
    st = lax.dot_general(cysy, img2, (((1,), (1,)), ((), ())),
                         preferred_element_type=jnp.float32,
                         precision=mxu_precision)
    t_r = st[:kblk]
    t_n = st[kblk:]

    cxt = jnp.concatenate([cx] * n_coils, axis=1)
    sxt = jnp.concatenate([sx] * n_coils, axis=1)
    m_r = cxt * t_r - sxt * t_n
    m_n = cxt * t_n + sxt * t_r

    jj = lax.broadcasted_iota(jnp.int32, (n_coils * _IM, n_coils), 0)
    bb = lax.broadcasted_iota(jnp.int32, (n_coils * _IM, n_coils), 1)
    g = (jj // _IM == bb).astype(jnp.float32)
    m = jnp.concatenate([m_r, m_n], axis=0)
    res = lax.dot_general(g, m, (((0,), (1,)), ((), ())),
                          preferred_element_type=jnp.float32,
                          precision=mxu_precision)
    outr_ref[...] = res[:, :kblk]
    outi_ref[...] = -res[:, kblk:]


def kernel(img, trj):
    n = img.shape[0]
    n_coils = img.shape[1]
    n_k = trj.shape[1]
    img2 = img.reshape(n_coils * _IM, _IM)
    trj2 = trj.reshape(n_k, 2)

    body = functools.partial(_nufft_block_kernel, n_coils=n_coils,
                             mxu_precision=lax.Precision.DEFAULT)
    outr, outi = pl.pallas_call(
        body,
        grid=(n_k // _KBLK,),
        in_specs=[
            pl.BlockSpec((_KBLK, 2), lambda i: (i, 0)),
            pl.BlockSpec((n_coils * _IM, _IM), lambda i: (0, 0)),
        ],
        out_specs=[
            pl.BlockSpec((n_coils, _KBLK), lambda i: (0, i)),
            pl.BlockSpec((n_coils, _KBLK), lambda i: (0, i)),
        ],
        out_shape=[jax.ShapeDtypeStruct((n_coils, n_k), jnp.float32)] * 2,
    )(trj2, img2)
    return lax.complex(outr, outi).reshape(n, n_coils, n_k)


# speedup vs baseline: 1.1091x; 1.0001x over previous
import functools
import math

import jax
import jax.numpy as jnp
from jax import lax
from jax.experimental import pallas as pl

_IM = 64
_KBLK = 2048

_COS_C = [(-1.0) ** j * (2.0 * math.pi) ** (2 * j) / math.factorial(2 * j)
          for j in range(6)]
_SIN_C = [(-1.0) ** j * (2.0 * math.pi) ** (2 * j + 1) / math.factorial(2 * j + 1)
          for j in range(5)]


def _sincos_cycles(ph):
    n = jnp.round(4.0 * ph)
    z = ph - 0.25 * n
    y = z * z
    cp = _COS_C[5]
    for coef in (_COS_C[4], _COS_C[3], _COS_C[2], _COS_C[1], _COS_C[0]):
        cp = cp * y + coef
    sp = _SIN_C[4]
    for coef in (_SIN_C[3], _SIN_C[2], _SIN_C[1], _SIN_C[0]):
        sp = sp * y + coef
    sp = sp * z
    i = n.astype(jnp.int32)
    bit0 = (i & 1) == 1
    c_sign = ((i ^ (i >> 1)) & 1) << 31
    s_sign = ((i >> 1) & 1) << 31
    c_abs = jnp.where(bit0, sp, cp)
    s_abs = jnp.where(bit0, cp, sp)
    c = lax.bitcast_convert_type(
        lax.bitcast_convert_type(c_abs, jnp.int32) ^ c_sign, jnp.float32)
    s = lax.bitcast_convert_type(
        lax.bitcast_convert_type(s_abs, jnp.int32) ^ s_sign, jnp.float32)
    return c, s


def _nufft_block_kernel(trj_ref, img_t_ref, outr_ref, outi_ref, *, n_coils,
                        mxu_precision):
    kblk = trj_ref.shape[0]
    trj = trj_ref[...]
    trj = trj.astype(jnp.bfloat16).astype(jnp.float32)
    kx = trj[:, 0:1]
    ky = trj[:, 1:2]

    col = lax.broadcasted_iota(jnp.int32, (1, 2 * _IM), 1)
    rv2 = ((col % _IM) - (_IM // 2)).astype(jnp.float32) * (1.0 / _IM)
    kxy = jnp.where(col < _IM, kx, ky)
    ph = kxy * rv2
    c, s = _sincos_cycles(ph)
    cx = c[:, :_IM]
    sx = s[:, :_IM]
    cysy = jnp.concatenate([c[:, _IM:], s[:, _IM:]], axis=0)

    img2 = img_t_ref[...]
    st = lax.dot_general(cysy, img2, (((1,), (1,)), ((), ())),
                         preferred_element_type=jnp.float32,
                         precision=mxu_precision)
    t_r = st[:kblk]
    t_n = st[kblk:]

    cxt = jnp.concatenate([cx] * n_coils, axis=1)
    sxt = jnp.concatenate([sx] * n_coils, axis=1)
    m_r = cxt * t_r - sxt * t_n
    m_n = cxt * t_n + sxt * t_r

    jj = lax.broadcasted_iota(jnp.int32, (n_coils * _IM, n_coils), 0)
    bb = lax.broadcasted_iota(jnp.int32, (n_coils * _IM, n_coils), 1)
    g = (jj // _IM == bb).astype(jnp.float32)
    m = jnp.concatenate([m_r, m_n], axis=0)
    res = lax.dot_general(g, m, (((0,), (1,)), ((), ())),
                          preferred_element_type=jnp.float32,
                          precision=mxu_precision)
    outr_ref[...] = res[:, :kblk]
    outi_ref[...] = -res[:, kblk:]


def kernel(img, trj):
    n = img.shape[0]
    n_coils = img.shape[1]
    n_k = trj.shape[1]
    img2 = img.reshape(n_coils * _IM, _IM)
    trj2 = trj.reshape(n_k, 2)

    body = functools.partial(_nufft_block_kernel, n_coils=n_coils,
                             mxu_precision=lax.Precision.DEFAULT)
    outr, outi = pl.pallas_call(
        body,
        grid=(n_k // _KBLK,),
        in_specs=[
            pl.BlockSpec((_KBLK, 2), lambda i: (i, 0)),
            pl.BlockSpec((n_coils * _IM, _IM), lambda i: (0, 0)),
        ],
        out_specs=[
            pl.BlockSpec((n_coils, _KBLK), lambda i: (0, i)),
            pl.BlockSpec((n_coils, _KBLK), lambda i: (0, i)),
        ],
        out_shape=[jax.ShapeDtypeStruct((n_coils, n_k), jnp.float32)] * 2,
    )(trj2, img2)
    return lax.complex(outr, outi).reshape(n, n_coils, n_k)
